# baseline (speedup 1.0000x reference)
import functools

import jax
import jax.numpy as jnp
from jax import lax
from jax.experimental import pallas as pl
from jax.experimental.pallas import tpu as pltpu
from jax.experimental.pallas import tpu_sc as plsc

B = 1024
DIM = 512
MAXN = 64
NC, NS, L = 2, 16, 16
NW = NC * NS
BPW = B // NW
NG = MAXN // L
DC = DIM // L
NA = 8


def _take_rows(table, idx):
  dnums = lax.GatherDimensionNumbers(
      offset_dims=tuple(range(1, table.ndim)),
      collapsed_slice_dims=(0,),
      start_index_map=(0,))
  return lax.gather(table, idx[:, None], dnums,
                    (1,) + table.shape[1:],
                    mode=lax.GatherScatterMode.PROMISE_IN_BOUNDS)


def _make(cnt_r, interpret=False):
  mesh = plsc.VectorSubcoreMesh(core_axis_name="c", subcore_axis_name="s",
                                num_cores=NC, num_subcores=NS)

  @functools.partial(
      pl.kernel,
      out_type=jax.ShapeDtypeStruct((B, DIM), jnp.float32),
      mesh=mesh,
      scratch_types=[
          pltpu.VMEM((BPW, MAXN * 2), jnp.int32),
          pltpu.VMEM((BPW, MAXN), jnp.float32),
          pltpu.VMEM((cnt_r,), jnp.float32),
          pltpu.VMEM((BPW, MAXN), jnp.float32),
          pltpu.VMEM((MAXN, DIM), jnp.float32),
          pltpu.VMEM((MAXN, DIM), jnp.float32),
          pltpu.VMEM((MAXN, DIM), jnp.float32),
          pltpu.VMEM((DIM,), jnp.float32),
          pltpu.VMEM((DIM,), jnp.float32),
          pltpu.VMEM((DIM,), jnp.float32),
          pltpu.SemaphoreType.DMA,
          pltpu.SemaphoreType.DMA,
          pltpu.SemaphoreType.DMA,
          pltpu.SemaphoreType.DMA,
          pltpu.SemaphoreType.DMA,
          pltpu.SemaphoreType.DMA,
      ],
      compiler_params=pltpu.CompilerParams(needs_layout_passes=False),
      interpret=interpret,
  )
  def k(nei_hbm, den_hbm, corr_hbm, emb_hbm, out_hbm,
        nei_v, tw_v, corr_v, w_v, rows0, rows1, rows2, o0, o1, o2,
        sem0, sem1, sem2, os0, os1, os2):
    wid = lax.axis_index("s") * NC + lax.axis_index("c")
    base = wid * BPW
    pltpu.sync_copy(nei_hbm.at[pl.ds(base, BPW)], nei_v)
    pltpu.sync_copy(den_hbm.at[pl.ds(base, BPW)], tw_v)
    pltpu.sync_copy(corr_hbm, corr_v)

    lane = lax.iota(jnp.int32, L)
    zero16 = jnp.zeros((L,), jnp.int32)
    one16 = jnp.full((L,), 1, jnp.int32)

    def weights(b):
      b16 = zero16 + b
      pend = []
      for g in range(NG):
        colr = g * (L * 2) + lane * 2
        rid = plsc.load_gather(nei_v, [b16, colr])
        eid = plsc.load_gather(nei_v, [b16, colr + one16])
        num = plsc.load_gather(corr_v, [rid])
        den = tw_v[b, pl.ds(g * L, L)]
        w_v[b, pl.ds(g * L, L)] = num / den
        pend.append((g, eid))
        if g >= 1:
          for gg, e in pend:
            nei_v[b, pl.ds(gg * L, L)] = e
          pend = []

    ring = ((rows0, sem0), (rows1, sem1), (rows2, sem2))
    def eids_of(b):
      return nei_v.at[b, pl.ds(0, MAXN)]
    for p in range(3):
      weights(p)
      pltpu.async_copy(emb_hbm.at[eids_of(p)], ring[p][0], ring[p][1])

    outs = ((o0, os0), (o1, os1), (o2, os2))

    def compute(b, rows_p, o_v):
      wg = [w_v[b, pl.ds(g * L, L)] for g in range(NG)]
      def dbody(dc, _):
        accs = [jnp.zeros((L,), jnp.float32) for _ in range(NA)]
        for n in range(MAXN):
          accs[n % NA] = (accs[n % NA]
                          + wg[n // L][n % L] * rows_p[n, pl.ds(dc * L, L)])
        while len(accs) > 1:
          accs = [a + c for a, c in zip(accs[0::2], accs[1::2])]
        o_v[pl.ds(dc * L, L)] = accs[0]
        return 0
      lax.fori_loop(0, DC, dbody, 0)

    def step(b, rows_p, sem_p, o_v, osem_p, refill):
      pltpu.make_async_copy(emb_hbm.at[eids_of(b)], rows_p, sem_p).wait()
      @pl.when(b >= 3)
      def _():
        pltpu.make_async_copy(o_v, out_hbm.at[base + b - 3], osem_p).wait()
      compute(b, rows_p, o_v)
      pltpu.async_copy(o_v, out_hbm.at[base + b], osem_p)
      if refill:
        weights(b + 3)
        pltpu.async_copy(emb_hbm.at[eids_of(b + 3)], rows_p, sem_p)

    def mbody(bb, _):
      for p in range(3):
        step(bb * 3 + p, ring[p][0], ring[p][1], outs[p][0], outs[p][1], True)
      return 0
    lax.fori_loop(0, BPW // 3 - 1, mbody, 0)
    for b in range(BPW - 5, BPW):
      step(b, ring[b % 3][0], ring[b % 3][1],
           outs[b % 3][0], outs[b % 3][1], b + 3 < BPW)
    for b in range(BPW - 3, BPW):
      pltpu.make_async_copy(outs[b % 3][0], out_hbm.at[base + b],
                            outs[b % 3][1]).wait()

  return k


@jax.jit
def kernel(hp, rp, tp, hn, rn, tn, e_emb, train_w, corr, train_g):
  del rp, tp, hn, rn, tn
  k = _make(corr.shape[0])
  nei = _take_rows(train_g.astype(jnp.int32), hp)
  den = _take_rows(train_w, hp)
  return k(nei.reshape(B, MAXN * 2), den, corr, e_emb)
